# baseline (speedup 1.0000x reference)
import jax
import jax.numpy as jnp
from jax.experimental import pallas as pl


def _dist_kernel(x_ref, w_ref, out_ref):
    x = x_ref[:]
    w = w_ref[:]
    xw = jax.lax.dot_general(
        x, w,
        dimension_numbers=(((1,), (1,)), ((), ())),
        preferred_element_type=jnp.float32,
        precision=jax.lax.Precision.DEFAULT,
    )
    x2 = jnp.sum(x * x, axis=1, keepdims=True)
    w2 = jnp.sum(w * w, axis=1, keepdims=True).T
    out_ref[:] = (x2 + w2) - 2.0 * xw


def kernel(x, weights):
    B, D = x.shape
    R, C, D2 = weights.shape
    N = R * C
    w = weights.reshape(N, D2)
    NBLK = 2
    BB = B // NBLK
    out = pl.pallas_call(
        _dist_kernel,
        grid=(NBLK,),
        in_specs=[
            pl.BlockSpec((BB, D), lambda i: (i, 0)),
            pl.BlockSpec((N, D2), lambda i: (0, 0)),
        ],
        out_specs=pl.BlockSpec((BB, N), lambda i: (i, 0)),
        out_shape=jax.ShapeDtypeStruct((B, N), jnp.float32),
    )(x, w)
    return out.reshape(B, R, C)


# speedup vs baseline: 1.1053x; 1.1053x over previous
import jax
import jax.numpy as jnp
from jax.experimental import pallas as pl


def _dist_kernel(x_ref, w_ref, out_ref):
    x = x_ref[:]
    w = w_ref[:]
    xw = jax.lax.dot_general(
        x, w,
        dimension_numbers=(((1,), (1,)), ((), ())),
        preferred_element_type=jnp.float32,
        precision=jax.lax.Precision.DEFAULT,
    )
    x2 = jnp.sum(x * x, axis=1, keepdims=True)
    w2 = jnp.sum(w * w, axis=1, keepdims=True).T
    out_ref[:] = (x2 + w2) - 2.0 * xw


def kernel(x, weights):
    B, D = x.shape
    R, C, D2 = weights.shape
    N = R * C
    w = weights.reshape(N, D2)
    NBLK = 2
    NB = N // NBLK
    out = pl.pallas_call(
        _dist_kernel,
        grid=(NBLK,),
        in_specs=[
            pl.BlockSpec((B, D), lambda i: (0, 0)),
            pl.BlockSpec((NB, D2), lambda i: (i, 0)),
        ],
        out_specs=pl.BlockSpec((B, NB), lambda i: (0, i)),
        out_shape=jax.ShapeDtypeStruct((B, N), jnp.float32),
    )(x, w)
    return out.reshape(B, R, C)
